# baseline (speedup 1.0000x reference)
import jax
import jax.numpy as jnp
from jax.experimental import pallas as pl
from jax.experimental.pallas import tpu as pltpu

_MAX_LEN = 200
_EMBED_DIM = 64
_FLAT = _MAX_LEN * _EMBED_DIM
_BB = 256
_NCHUNK = 4096 // _BB


def _body(pe_ref, o_hbm, scratch, sems):
    scratch[...] = jnp.broadcast_to(pe_ref[...], scratch.shape)
    first = pltpu.make_async_copy(scratch, o_hbm.at[pl.ds(0, _BB), :], sems.at[0])
    first.start()
    first.wait()
    done = 1
    while done < _NCHUNK:
        n = min(done, _NCHUNK - done)
        copies = [
            pltpu.make_async_copy(
                o_hbm.at[pl.ds(j * _BB, _BB), :],
                o_hbm.at[pl.ds((done + j) * _BB, _BB), :],
                sems.at[j],
            )
            for j in range(n)
        ]
        for c in copies:
            c.start()
        for c in copies:
            c.wait()
        done += n


def kernel(x, pe_weight):
    batch = x.shape[0]
    pe_flat = pe_weight.reshape(1, _FLAT)
    out = pl.pallas_call(
        _body,
        in_specs=[pl.BlockSpec(memory_space=pltpu.MemorySpace.VMEM)],
        out_specs=pl.BlockSpec(memory_space=pltpu.MemorySpace.HBM),
        out_shape=jax.ShapeDtypeStruct((batch, _FLAT), jnp.float32),
        scratch_shapes=[
            pltpu.VMEM((_BB, _FLAT), jnp.float32),
            pltpu.SemaphoreType.DMA((_NCHUNK // 2,)),
        ],
    )(pe_flat)
    return out.reshape(batch, _MAX_LEN, _EMBED_DIM)


# speedup vs baseline: 25.6071x; 25.6071x over previous
import jax
import jax.numpy as jnp
from jax.experimental import pallas as pl
from jax.experimental.pallas import tpu as pltpu

_MAX_LEN = 200
_EMBED_DIM = 64
_FLAT = _MAX_LEN * _EMBED_DIM
_BB = 256
_NCHUNK = 4096 // _BB
_NBUF = 4


def _body(pe_ref, o_hbm, b0, b1, b2, b3, sems):
    bufs = [b0, b1, b2, b3]
    for b in bufs:
        b[...] = jnp.broadcast_to(pe_ref[...], b.shape)
    copies = [
        pltpu.make_async_copy(
            bufs[i % _NBUF], o_hbm.at[pl.ds(i * _BB, _BB), :], sems.at[i]
        )
        for i in range(_NCHUNK)
    ]
    for c in copies:
        c.start()
    for c in copies:
        c.wait()


def kernel(x, pe_weight):
    batch = x.shape[0]
    pe_flat = pe_weight.reshape(1, _FLAT)
    out = pl.pallas_call(
        _body,
        in_specs=[pl.BlockSpec(memory_space=pltpu.MemorySpace.VMEM)],
        out_specs=pl.BlockSpec(memory_space=pltpu.MemorySpace.HBM),
        out_shape=jax.ShapeDtypeStruct((batch, _FLAT), jnp.float32),
        scratch_shapes=[
            pltpu.VMEM((_BB, _FLAT), jnp.float32),
            pltpu.VMEM((_BB, _FLAT), jnp.float32),
            pltpu.VMEM((_BB, _FLAT), jnp.float32),
            pltpu.VMEM((_BB, _FLAT), jnp.float32),
            pltpu.SemaphoreType.DMA((_NCHUNK,)),
        ],
    )(pe_flat)
    return out.reshape(batch, _MAX_LEN, _EMBED_DIM)
